# baseline (speedup 1.0000x reference)
import jax
import jax.numpy as jnp
from jax import lax
from jax.experimental import pallas as pl
from jax.experimental.pallas import tpu as pltpu, tpu_sc as plsc

D = 64
NC, NS = 2, 16
NW = NC * NS
BS = 128
NB = 4
SCALE = 8.0


def _body(xt_hbm, tab_hbm, out_hbm, *scratch):
    idxs = scratch[:NB]
    sidxs = scratch[NB:2 * NB]
    cols = scratch[2 * NB:3 * NB]
    gbufs = scratch[3 * NB:4 * NB]
    obufs = scratch[4 * NB:5 * NB]
    isems = scratch[5 * NB:6 * NB]
    gsems = scratch[6 * NB:7 * NB]
    osems = scratch[7 * NB:8 * NB]
    wid = lax.axis_index("s") * NC + lax.axis_index("c")
    nt, ntok = xt_hbm.shape
    nj = ntok // BS
    nch = nt * nj // NW
    c0 = wid * nch

    def tj(c):
        t = (c0 + c) // nj
        return t, (c0 + c) - t * nj

    def idx_dma(c, s):
        t, j = tj(c)
        return pltpu.make_async_copy(
            xt_hbm.at[t, pl.ds(j * BS, BS)], idxs[s], isems[s])

    def gat_dma(c, s):
        del c
        return pltpu.make_async_copy(tab_hbm.at[sidxs[s]], gbufs[s], gsems[s])

    def shift(s):
        for g in range(BS // 16):
            sl = pl.ds(g * 16, 16)
            iv = idxs[s][sl]
            sidxs[s][sl] = lax.shift_right_logical(iv, 1)
            cols[s][sl] = (iv & 1) * D

    class _Multi:
        def __init__(self, parts):
            self.parts = parts

        def start(self):
            for p in self.parts:
                p.start()

        def wait(self):
            for p in self.parts:
                p.wait()

    def out_dma(c, s):
        t, j = tj(c)
        return _Multi([
            pltpu.make_async_copy(
                obufs[s].at[pl.ds(g * 8, 8), :],
                out_hbm.at[t, pl.ds(g * 8, 8), pl.ds(j * BS, BS)],
                osems[s])
            for g in range(D // 8)
        ])

    rows = [lax.iota(jnp.int32, 16) + g * 16 for g in range(BS // 16)]

    for c in range(NB):
        idx_dma(c, c % NB).start()
    for c in range(NB - 1):
        idx_dma(c, c % NB).wait()
        shift(c % NB)
        gat_dma(c, c % NB).start()
    idx_dma(NB, 0).start()

    def step(o, carry):
        for b in range(NB):
            c = o * NB + b

            @pl.when(c >= NB)
            def _():
                out_dma(c - NB, b).wait()

            f = c + NB - 1
            fs = (b + NB - 1) % NB

            @pl.when(f < nch)
            def _():
                idx_dma(f, fs).wait()
                shift(fs)
                gat_dma(f, fs).start()

            f2 = c + NB + 1
            fs2 = (b + 1) % NB

            @pl.when(f2 < nch)
            def _():
                idx_dma(f2, fs2).start()

            gat_dma(c, b).wait()
            gbuf, obuf = gbufs[b], obufs[b]
            cvecs = [cols[b][pl.ds(g * 16, 16)] for g in range(BS // 16)]

            @plsc.parallel_loop(0, D, step=1, unroll=2)
            def transform(d):
                vals = [plsc.load_gather(gbuf, [rows[g], cvecs[g] + d])
                        for g in range(BS // 16)]
                for g in range(BS // 16):
                    obuf[d, pl.ds(g * 16, 16)] = vals[g]

            out_dma(c, b).start()
        return carry

    lax.fori_loop(0, nch // NB, step, 0)
    for k in range(NB):
        c = nch - NB + k
        out_dma(c, c % NB).wait()


def _tbody(tt_hbm, out_hbm, *scratch):
    ibufs = scratch[:NB]
    obufs = scratch[NB:2 * NB]
    isems = scratch[2 * NB:3 * NB]
    osems = scratch[3 * NB:4 * NB]
    wid = lax.axis_index("s") * NC + lax.axis_index("c")
    nv = tt_hbm.shape[1]
    nblk = nv // BS // NW
    j0 = wid * nblk

    class _Multi:
        def __init__(self, parts):
            self.parts = parts

        def start(self):
            for p in self.parts:
                p.start()

        def wait(self):
            for p in self.parts:
                p.wait()

    def in_dma(k, s):
        return _Multi([
            pltpu.make_async_copy(
                tt_hbm.at[pl.ds(g * 8, 8), pl.ds((j0 + k) * BS, BS)],
                ibufs[s].at[pl.ds(g * 8, 8), :],
                isems[s])
            for g in range(D // 8)
        ])

    def out_dma(k, s):
        return pltpu.make_async_copy(
            obufs[s], out_hbm.at[pl.ds((j0 + k) * D, D), :], osems[s])

    dvecs = [lax.iota(jnp.int32, 16) + q * 16 for q in range(D // 16)]

    for k in range(NB - 1):
        in_dma(k, k).start()

    def step(o, carry):
        for b in range(NB):
            k = o * NB + b

            @pl.when(k >= NB)
            def _():
                out_dma(k - NB, b).wait()

            f = k + NB - 1
            fs = (b + NB - 1) % NB

            @pl.when(f < nblk)
            def _():
                in_dma(f, fs).start()

            in_dma(k, b).wait()
            ibuf, obuf = ibufs[b], obufs[b]

            @plsc.parallel_loop(0, D, step=1, unroll=2)
            def transpose(w):
                vvs = [jnp.full((16,), 2 * w + h, jnp.int32) for h in range(2)]
                vals = [plsc.load_gather(ibuf, [dvecs[q], vvs[h]])
                        for h in range(2) for q in range(D // 16)]
                for h in range(2):
                    for q in range(D // 16):
                        obuf[w, pl.ds(h * D + q * 16, 16)] = (
                            vals[h * (D // 16) + q] * SCALE)

            out_dma(k, b).start()
        return carry

    lax.fori_loop(0, nblk // NB, step, 0)
    for k2 in range(NB):
        out_dma(nblk - NB + k2, (nblk - NB + k2) % NB).wait()

    full = nv // BS
    rem = nv % BS
    ibt, obt = scratch[4 * NB], scratch[4 * NB + 1]

    for i in range(full - NW * nblk):
        v0 = (NW * nblk + i) * BS

        @pl.when(wid == i)
        def _():
            cp_in = pltpu.make_async_copy(
                tt_hbm.at[:, pl.ds(v0, BS)], ibufs[0], isems[0])
            cp_in.start()
            cp_in.wait()
            ibuf, obuf = ibufs[0], obufs[0]

            @plsc.parallel_loop(0, D, step=1, unroll=4)
            def transpose_tail(w):
                for h in range(2):
                    vv = jnp.full((16,), 2 * w + h, jnp.int32)
                    for q in range(D // 16):
                        vals = plsc.load_gather(ibuf, [dvecs[q], vv])
                        obuf[w, pl.ds(h * D + q * 16, 16)] = vals * SCALE

            cp_out = pltpu.make_async_copy(
                obuf, out_hbm.at[pl.ds(v0 // 2, D), :], osems[0])
            cp_out.start()
            cp_out.wait()

    if rem:
        @pl.when(wid == full - NW * nblk)
        def _():
            cp_in = pltpu.make_async_copy(
                tt_hbm.at[:, pl.ds(nv - rem, rem)], ibt, isems[0])
            cp_in.start()
            cp_in.wait()

            @plsc.parallel_loop(0, rem // 2, step=1, unroll=4)
            def transpose_rem(w):
                for h in range(2):
                    vv = jnp.full((16,), 2 * w + h, jnp.int32)
                    for q in range(D // 16):
                        vals = plsc.load_gather(ibt, [dvecs[q], vv])
                        obt[w, pl.ds(h * D + q * 16, 16)] = vals * SCALE

            cp_out = pltpu.make_async_copy(
                obt, out_hbm.at[pl.ds((nv - rem) // 2, rem // 2), :], osems[0])
            cp_out.start()
            cp_out.wait()


@jax.jit
def _format(tt):
    nv = tt.shape[1]
    mesh = plsc.VectorSubcoreMesh(core_axis_name="c", subcore_axis_name="s")
    return pl.kernel(
        _tbody,
        out_type=jax.ShapeDtypeStruct((nv // 2, 2 * D), jnp.float32),
        mesh=mesh,
        scratch_types=(
            [pltpu.VMEM((D, BS), jnp.float32) for _ in range(2 * NB)]
            + [pltpu.SemaphoreType.DMA for _ in range(2 * NB)]
            + [pltpu.VMEM((D, BS // 2), jnp.float32),
               pltpu.VMEM((BS // 4, BS), jnp.float32)]
        ),
        compiler_params=pltpu.CompilerParams(
            use_tc_tiling_on_sc=True, needs_layout_passes=False),
    )(tt)


@jax.jit
def _embed(xt, tab):
    nt, ntok = xt.shape
    mesh = plsc.VectorSubcoreMesh(core_axis_name="c", subcore_axis_name="s")
    return pl.kernel(
        _body,
        out_type=jax.ShapeDtypeStruct((nt, D, ntok), jnp.float32),
        mesh=mesh,
        scratch_types=(
            [pltpu.VMEM((BS,), jnp.int32) for _ in range(3 * NB)]
            + [pltpu.VMEM((BS, 2 * D), jnp.float32) for _ in range(NB)]
            + [pltpu.VMEM((D, BS), jnp.float32) for _ in range(NB)]
            + [pltpu.SemaphoreType.DMA for _ in range(3 * NB)]
        ),
        compiler_params=pltpu.CompilerParams(
            use_tc_tiling_on_sc=True, needs_layout_passes=False),
    )(xt, tab)


def kernel(x, table):
    tab = _format(table.T)
    out_phys = _embed(x.T, tab)
    return jnp.transpose(out_phys, (2, 0, 1))


# speedup vs baseline: 1.5061x; 1.5061x over previous
import jax
import jax.numpy as jnp
from jax import lax
from jax.experimental import pallas as pl
from jax.experimental.pallas import tpu as pltpu, tpu_sc as plsc

D = 64
NC, NS = 2, 16
NW = NC * NS
CH = 128
NBUF = 5
PRE = 3
SCALE = 8.0


def _body(idx_hbm, table_hbm, out_hbm, idx_v, *scratch):
    bufs = scratch[:NBUF]
    gsems = scratch[NBUF:2 * NBUF]
    osems = scratch[2 * NBUF:3 * NBUF]
    wid = lax.axis_index("s") * NC + lax.axis_index("c")
    nch = idx_hbm.shape[1]
    base = wid * nch * CH

    def gather(c, b, wait=False):
        d = pltpu.make_async_copy(table_hbm.at[idx_v.at[c]], bufs[b], gsems[b])
        d.wait() if wait else d.start()

    def store(c, b, wait=False):
        d = pltpu.make_async_copy(
            bufs[b], out_hbm.at[pl.ds(base + c * CH, CH)], osems[b])
        d.wait() if wait else d.start()

    pltpu.sync_copy(idx_hbm.at[wid], idx_v)
    for c in range(PRE):
        gather(c, c % NBUF)

    def step(o, carry):
        for b in range(NBUF):
            c = o * NBUF + b
            gather(c, b, wait=True)

            @plsc.parallel_loop(0, CH, step=1, unroll=4)
            def scale(r):
                for t in range(D // 16):
                    sl = pl.ds(t * 16, 16)
                    bufs[b][r, sl] = bufs[b][r, sl] * SCALE

            store(c, b)
            f = c + PRE
            fb = (b + PRE) % NBUF

            @pl.when(f < nch)
            def _():
                @pl.when(f >= NBUF)
                def _():
                    store(f - NBUF, fb, wait=True)
                gather(f, fb)
        return carry

    lax.fori_loop(0, nch // NBUF, step, 0)
    for k in range(NBUF):
        c = nch - NBUF + k
        store(c, c % NBUF, wait=True)


@jax.jit
def _embed(xf, table):
    b = xf.shape[0]
    nch = b // (NW * CH)
    idx3 = xf.reshape(NW, nch, CH)
    mesh = plsc.VectorSubcoreMesh(core_axis_name="c", subcore_axis_name="s")
    return pl.kernel(
        _body,
        out_type=jax.ShapeDtypeStruct((b, D), jnp.float32),
        mesh=mesh,
        scratch_types=(
            [pltpu.VMEM((nch, CH), jnp.int32)]
            + [pltpu.VMEM((CH, D), jnp.float32) for _ in range(NBUF)]
            + [pltpu.SemaphoreType.DMA for _ in range(2 * NBUF)]
        ),
        compiler_params=pltpu.CompilerParams(use_tc_tiling_on_sc=False),
    )(idx3, table)


def kernel(x, table):
    s, t = x.shape
    out = _embed(x.reshape(s * t), table)
    return out.reshape(s, t, D)
